# baseline (speedup 1.0000x reference)
import functools

import jax
import jax.numpy as jnp
from jax import lax
from jax.experimental import pallas as pl
from jax.experimental.pallas import tpu as pltpu
from jax.experimental.pallas import tpu_sc as plsc

_B, _N, _K, _D, _H, _C = 2, 10000, 16, 128, 128, 64
_M = _B * _N
_NW = 32
_MP = 20480
_HALF = _MP // 2
_CCH = 1024
_NCHK = _HALF // _CCH
_GRP = _CCH // 16
_W = _H // 2 // 16
_BN = 2048
_MASK = -65536


def _sc_aggregate_t(hTp, idxc):
    mesh = plsc.VectorSubcoreMesh(core_axis_name="c", subcore_axis_name="s")

    @functools.partial(
        pl.kernel,
        out_type=jax.ShapeDtypeStruct((_H, _MP), jnp.float32),
        mesh=mesh,
        scratch_types=[
            pltpu.VMEM((_W * _MP,), jnp.int32),
            pltpu.VMEM((_K, _CCH), jnp.int32),
            pltpu.VMEM((2 * _W, _CCH), jnp.float32),
        ],
        compiler_params=pltpu.CompilerParams(needs_layout_passes=False),
    )
    def agg_kernel(tab_hbm, idx_hbm, out_hbm, tab_v, idx_v, out_v):
        wid = lax.axis_index("s") * 2 + lax.axis_index("c")
        wr = (wid % 16) * _W
        half = wid // 16
        node0 = half * _HALF
        pltpu.sync_copy(tab_hbm.at[pl.ds(wr * _MP, _W * _MP)], tab_v)

        for chunk in range(_NCHK):
            pltpu.sync_copy(idx_hbm.at[half, chunk], idx_v)

            def group(g):
                col = pl.ds(g * 16, 16)
                acc_lo = [None] * _W
                acc_hi = [None] * _W
                for kk in range(_K):
                    iv = idx_v[kk, col]
                    for w in range(_W):
                        word = plsc.load_gather(tab_v, [iv + w * _MP])
                        lo = plsc.bitcast(word << 16, jnp.float32)
                        hi = plsc.bitcast(word & _MASK, jnp.float32)
                        if kk == 0:
                            acc_lo[w], acc_hi[w] = lo, hi
                        else:
                            acc_lo[w] = acc_lo[w] + lo
                            acc_hi[w] = acc_hi[w] + hi
                for w in range(_W):
                    out_v[w, col] = acc_lo[w]
                    out_v[_W + w, col] = acc_hi[w]

            plsc.parallel_loop(0, _GRP, unroll=2)(group)
            col0 = node0 + chunk * _CCH
            pltpu.sync_copy(
                out_v.at[pl.ds(0, _W)],
                out_hbm.at[pl.ds(wr, _W), pl.ds(col0, _CCH)])
            pltpu.sync_copy(
                out_v.at[pl.ds(_W, _W)],
                out_hbm.at[pl.ds(wr + 64, _W), pl.ds(col0, _CCH)])

    return agg_kernel(hTp, idxc)


def _pack_bf16_pairs(t):
    u = jax.lax.bitcast_convert_type(t, jnp.uint32)
    rne = lambda v: (v + jnp.uint32(0x7FFF) + ((v >> 16) & jnp.uint32(1))) >> 16
    word = rne(u[:64]) | (rne(u[64:]) << 16)
    return jax.lax.bitcast_convert_type(word, jnp.int32)


def _embed_body(x_ref, w_ref, ht_ref, pk_ref):
    x = x_ref[...]
    rs = jax.lax.dot_general(jnp.ones((1, _D), jnp.float32), x,
                             (((1,), (1,)), ((), ())),
                             preferred_element_type=jnp.float32)
    ht = jax.lax.dot_general(w_ref[...], x, (((0,), (1,)), ((), ())),
                             preferred_element_type=jnp.float32) / rs
    ht_ref[...] = ht
    pk_ref[...] = _pack_bf16_pairs(ht)


def _tc_embed_t(x, w):
    return pl.pallas_call(
        _embed_body,
        grid=(_MP // _BN,),
        in_specs=[
            pl.BlockSpec((_BN, _D), lambda i: (i, 0)),
            pl.BlockSpec((_D, _H), lambda i: (0, 0)),
        ],
        out_specs=[
            pl.BlockSpec((_H, _BN), lambda i: (0, i)),
            pl.BlockSpec((_H // 2, _BN), lambda i: (0, i)),
        ],
        out_shape=[
            jax.ShapeDtypeStruct((_H, _MP), jnp.float32),
            jax.ShapeDtypeStruct((_H // 2, _MP), jnp.int32),
        ],
    )(x, w)


def _layer_body(aggt_ref, ht_ref, ivl_ref, ws_ref, bs_ref, o_ref, pk_ref):
    a = aggt_ref[...] * ivl_ref[...]
    hn = jnp.maximum(
        jax.lax.dot_general(ws_ref[...], a, (((0,), (0,)), ((), ())),
                            preferred_element_type=jnp.float32)
        + jax.lax.dot_general(bs_ref[...], ht_ref[...],
                              (((0,), (0,)), ((), ())),
                              preferred_element_type=jnp.float32),
        0.0)
    o_ref[...] = hn
    pk_ref[...] = _pack_bf16_pairs(hn)


def _tc_layer_t(aggt, ht, ivl, ws, bs):
    return pl.pallas_call(
        _layer_body,
        grid=(_MP // _BN,),
        in_specs=[
            pl.BlockSpec((_H, _BN), lambda i: (0, i)),
            pl.BlockSpec((_H, _BN), lambda i: (0, i)),
            pl.BlockSpec((1, _BN), lambda i: (0, i)),
            pl.BlockSpec((_H, _H), lambda i: (0, 0)),
            pl.BlockSpec((_H, _H), lambda i: (0, 0)),
        ],
        out_specs=[
            pl.BlockSpec((_H, _BN), lambda i: (0, i)),
            pl.BlockSpec((_H // 2, _BN), lambda i: (0, i)),
        ],
        out_shape=[
            jax.ShapeDtypeStruct((_H, _MP), jnp.float32),
            jax.ShapeDtypeStruct((_H // 2, _MP), jnp.int32),
        ],
    )(aggt, ht, ivl, ws, bs)


def _last_body(aggt_ref, ht_ref, ivl_ref, ws_ref, bs_ref,
               w1_ref, b1_ref, w2_ref, b2_ref, o_ref):
    a = aggt_ref[...] * ivl_ref[...]
    hn = jnp.maximum(
        jax.lax.dot_general(ws_ref[...], a, (((0,), (0,)), ((), ())),
                            preferred_element_type=jnp.float32)
        + jax.lax.dot_general(bs_ref[...], ht_ref[...],
                              (((0,), (0,)), ((), ())),
                              preferred_element_type=jnp.float32),
        0.0)
    zt = jnp.maximum(
        jax.lax.dot_general(w1_ref[...], hn, (((0,), (0,)), ((), ())),
                            preferred_element_type=jnp.float32)
        + b1_ref[...],
        0.0)
    lg = jax.lax.dot_general(w2_ref[...], zt, (((0,), (0,)), ((), ())),
                             preferred_element_type=jnp.float32) + b2_ref[...]
    m = jnp.max(lg, axis=0, keepdims=True)
    e = jnp.exp(lg - m)
    o_ref[...] = e / jnp.sum(e, axis=0, keepdims=True)


def _tc_last_t(aggt, ht, ivl, ws, bs, w1, b1, w2, b2):
    return pl.pallas_call(
        _last_body,
        grid=(_MP // _BN,),
        in_specs=[
            pl.BlockSpec((_H, _BN), lambda i: (0, i)),
            pl.BlockSpec((_H, _BN), lambda i: (0, i)),
            pl.BlockSpec((1, _BN), lambda i: (0, i)),
            pl.BlockSpec((_H, _H), lambda i: (0, 0)),
            pl.BlockSpec((_H, _H), lambda i: (0, 0)),
            pl.BlockSpec((_H, _H), lambda i: (0, 0)),
            pl.BlockSpec((_H, 1), lambda i: (0, 0)),
            pl.BlockSpec((_H, _C), lambda i: (0, 0)),
            pl.BlockSpec((_C, 1), lambda i: (0, 0)),
        ],
        out_specs=pl.BlockSpec((_C, _BN), lambda i: (0, i)),
        out_shape=jax.ShapeDtypeStruct((_C, _MP), jnp.float32),
    )(aggt, ht, ivl, ws, bs, w1, b1, w2, b2)


def kernel(vertex_feat, neighbors_idx, valid_lens, W_embed, Ws, Bs,
           Wc1, bc1, Wc2, bc2):
    x = vertex_feat.reshape(_M, _D)
    xp = jnp.pad(x, ((0, _MP - _M), (0, 0)), constant_values=1.0)

    offs = (jnp.arange(_B, dtype=jnp.int32) * _N)[:, None, None]
    idx = (neighbors_idx + offs).reshape(_M, _K)
    idx = jnp.pad(idx, ((0, _MP - _M), (0, 0)))
    idxc = idx.T.reshape(_K, 2, _NCHK, _CCH).transpose(1, 2, 0, 3)

    vl = jnp.maximum(valid_lens, 1).astype(jnp.float32).reshape(1, _M)
    ivl = jnp.pad(1.0 / vl, ((0, 0), (0, _MP - _M)), constant_values=1.0)

    nl = Ws.shape[0]
    ht, htp = _tc_embed_t(xp, W_embed)
    for l in range(nl - 1):
        aggt = _sc_aggregate_t(htp.reshape(-1), idxc)
        ht, htp = _tc_layer_t(aggt, ht, ivl, Ws[l], Bs[l])
    aggt = _sc_aggregate_t(htp.reshape(-1), idxc)
    probst = _tc_last_t(aggt, ht, ivl, Ws[nl - 1], Bs[nl - 1],
                        Wc1, bc1.reshape(_H, 1), Wc2, bc2.reshape(_C, 1))
    return probst[:, :_M].T.reshape(_B, _N, _C)


# speedup vs baseline: 1.1348x; 1.1348x over previous
import functools

import jax
import jax.numpy as jnp
from jax import lax
from jax.experimental import pallas as pl
from jax.experimental.pallas import tpu as pltpu
from jax.experimental.pallas import tpu_sc as plsc

_B, _N, _K, _D, _H, _C = 2, 10000, 16, 128, 128, 64
_M = _B * _N
_NW = 32
_MP = 20480
_HALF = _MP // 2
_CCH = 1024
_NCHK = _HALF // _CCH
_GRP = _CCH // 16
_W = _H // 2 // 16
_BN = 2048
_MASK = -65536


def _sc_aggregate_t(hTp, idxc):
    mesh = plsc.VectorSubcoreMesh(core_axis_name="c", subcore_axis_name="s")

    @functools.partial(
        pl.kernel,
        out_type=jax.ShapeDtypeStruct((_H, _MP), jnp.float32),
        mesh=mesh,
        scratch_types=[
            pltpu.VMEM((_W * _MP,), jnp.int32),
            pltpu.VMEM((_K, _CCH), jnp.int32),
            pltpu.VMEM((2 * _W, _CCH), jnp.float32),
        ],
        compiler_params=pltpu.CompilerParams(needs_layout_passes=False),
    )
    def agg_kernel(tab_hbm, idx_hbm, out_hbm, tab_v, idx_v, out_v):
        wid = lax.axis_index("s") * 2 + lax.axis_index("c")
        wr = (wid % 16) * _W
        half = wid // 16
        node0 = half * _HALF
        pltpu.sync_copy(tab_hbm.at[pl.ds(wr * _MP, _W * _MP)], tab_v)

        for chunk in range(_NCHK):
            pltpu.sync_copy(idx_hbm.at[half, chunk], idx_v)

            def group(g, carry):
                col = pl.ds(g * 16, 16)
                acc_lo = [None] * _W
                acc_hi = [None] * _W
                for kk in range(_K):
                    iv = idx_v[kk, col]
                    for w in range(_W):
                        word = plsc.load_gather(tab_v, [iv + w * _MP])
                        lo = plsc.bitcast(word << 16, jnp.float32)
                        hi = plsc.bitcast(word & _MASK, jnp.float32)
                        if kk == 0:
                            acc_lo[w], acc_hi[w] = lo, hi
                        else:
                            acc_lo[w] = acc_lo[w] + lo
                            acc_hi[w] = acc_hi[w] + hi
                for w in range(_W):
                    out_v[w, col] = acc_lo[w]
                    out_v[_W + w, col] = acc_hi[w]
                return carry

            lax.fori_loop(0, _GRP, group, 0)
            col0 = node0 + chunk * _CCH
            pltpu.sync_copy(
                out_v.at[pl.ds(0, _W)],
                out_hbm.at[pl.ds(wr, _W), pl.ds(col0, _CCH)])
            pltpu.sync_copy(
                out_v.at[pl.ds(_W, _W)],
                out_hbm.at[pl.ds(wr + 64, _W), pl.ds(col0, _CCH)])

    return agg_kernel(hTp, idxc)


def _pack_bf16_pairs(t):
    u = jax.lax.bitcast_convert_type(t, jnp.uint32)
    rne = lambda v: (v + jnp.uint32(0x7FFF) + ((v >> 16) & jnp.uint32(1))) >> 16
    word = rne(u[:64]) | (rne(u[64:]) << 16)
    return jax.lax.bitcast_convert_type(word, jnp.int32)


def _embed_body(x_ref, w_ref, ht_ref, pk_ref):
    x = x_ref[...]
    rs = jax.lax.dot_general(jnp.ones((1, _D), jnp.float32), x,
                             (((1,), (1,)), ((), ())),
                             preferred_element_type=jnp.float32)
    ht = jax.lax.dot_general(w_ref[...], x, (((0,), (1,)), ((), ())),
                             preferred_element_type=jnp.float32) / rs
    ht_ref[...] = ht
    pk_ref[...] = _pack_bf16_pairs(ht)


def _tc_embed_t(x, w):
    return pl.pallas_call(
        _embed_body,
        grid=(_MP // _BN,),
        in_specs=[
            pl.BlockSpec((_BN, _D), lambda i: (i, 0)),
            pl.BlockSpec((_D, _H), lambda i: (0, 0)),
        ],
        out_specs=[
            pl.BlockSpec((_H, _BN), lambda i: (0, i)),
            pl.BlockSpec((_H // 2, _BN), lambda i: (0, i)),
        ],
        out_shape=[
            jax.ShapeDtypeStruct((_H, _MP), jnp.float32),
            jax.ShapeDtypeStruct((_H // 2, _MP), jnp.int32),
        ],
    )(x, w)


def _layer_body(aggt_ref, ht_ref, ivl_ref, ws_ref, bs_ref, o_ref, pk_ref):
    a = aggt_ref[...] * ivl_ref[...]
    hn = jnp.maximum(
        jax.lax.dot_general(ws_ref[...], a, (((0,), (0,)), ((), ())),
                            preferred_element_type=jnp.float32)
        + jax.lax.dot_general(bs_ref[...], ht_ref[...],
                              (((0,), (0,)), ((), ())),
                              preferred_element_type=jnp.float32),
        0.0)
    o_ref[...] = hn
    pk_ref[...] = _pack_bf16_pairs(hn)


def _tc_layer_t(aggt, ht, ivl, ws, bs):
    return pl.pallas_call(
        _layer_body,
        grid=(_MP // _BN,),
        in_specs=[
            pl.BlockSpec((_H, _BN), lambda i: (0, i)),
            pl.BlockSpec((_H, _BN), lambda i: (0, i)),
            pl.BlockSpec((1, _BN), lambda i: (0, i)),
            pl.BlockSpec((_H, _H), lambda i: (0, 0)),
            pl.BlockSpec((_H, _H), lambda i: (0, 0)),
        ],
        out_specs=[
            pl.BlockSpec((_H, _BN), lambda i: (0, i)),
            pl.BlockSpec((_H // 2, _BN), lambda i: (0, i)),
        ],
        out_shape=[
            jax.ShapeDtypeStruct((_H, _MP), jnp.float32),
            jax.ShapeDtypeStruct((_H // 2, _MP), jnp.int32),
        ],
    )(aggt, ht, ivl, ws, bs)


def _last_body(aggt_ref, ht_ref, ivl_ref, ws_ref, bs_ref,
               w1_ref, b1_ref, w2_ref, b2_ref, o_ref):
    a = aggt_ref[...] * ivl_ref[...]
    hn = jnp.maximum(
        jax.lax.dot_general(ws_ref[...], a, (((0,), (0,)), ((), ())),
                            preferred_element_type=jnp.float32)
        + jax.lax.dot_general(bs_ref[...], ht_ref[...],
                              (((0,), (0,)), ((), ())),
                              preferred_element_type=jnp.float32),
        0.0)
    zt = jnp.maximum(
        jax.lax.dot_general(w1_ref[...], hn, (((0,), (0,)), ((), ())),
                            preferred_element_type=jnp.float32)
        + b1_ref[...],
        0.0)
    lg = jax.lax.dot_general(w2_ref[...], zt, (((0,), (0,)), ((), ())),
                             preferred_element_type=jnp.float32) + b2_ref[...]
    m = jnp.max(lg, axis=0, keepdims=True)
    e = jnp.exp(lg - m)
    o_ref[...] = e / jnp.sum(e, axis=0, keepdims=True)


def _tc_last_t(aggt, ht, ivl, ws, bs, w1, b1, w2, b2):
    return pl.pallas_call(
        _last_body,
        grid=(_MP // _BN,),
        in_specs=[
            pl.BlockSpec((_H, _BN), lambda i: (0, i)),
            pl.BlockSpec((_H, _BN), lambda i: (0, i)),
            pl.BlockSpec((1, _BN), lambda i: (0, i)),
            pl.BlockSpec((_H, _H), lambda i: (0, 0)),
            pl.BlockSpec((_H, _H), lambda i: (0, 0)),
            pl.BlockSpec((_H, _H), lambda i: (0, 0)),
            pl.BlockSpec((_H, 1), lambda i: (0, 0)),
            pl.BlockSpec((_H, _C), lambda i: (0, 0)),
            pl.BlockSpec((_C, 1), lambda i: (0, 0)),
        ],
        out_specs=pl.BlockSpec((_C, _BN), lambda i: (0, i)),
        out_shape=jax.ShapeDtypeStruct((_C, _MP), jnp.float32),
    )(aggt, ht, ivl, ws, bs, w1, b1, w2, b2)


def kernel(vertex_feat, neighbors_idx, valid_lens, W_embed, Ws, Bs,
           Wc1, bc1, Wc2, bc2):
    x = vertex_feat.reshape(_M, _D)
    xp = jnp.pad(x, ((0, _MP - _M), (0, 0)), constant_values=1.0)

    offs = (jnp.arange(_B, dtype=jnp.int32) * _N)[:, None, None]
    idx = (neighbors_idx + offs).reshape(_M, _K)
    idx = jnp.pad(idx, ((0, _MP - _M), (0, 0)))
    idxc = idx.T.reshape(_K, 2, _NCHK, _CCH).transpose(1, 2, 0, 3)

    vl = jnp.maximum(valid_lens, 1).astype(jnp.float32).reshape(1, _M)
    ivl = jnp.pad(1.0 / vl, ((0, 0), (0, _MP - _M)), constant_values=1.0)

    nl = Ws.shape[0]
    ht, htp = _tc_embed_t(xp, W_embed)
    for l in range(nl - 1):
        aggt = _sc_aggregate_t(htp.reshape(-1), idxc)
        ht, htp = _tc_layer_t(aggt, ht, ivl, Ws[l], Bs[l])
    aggt = _sc_aggregate_t(htp.reshape(-1), idxc)
    probst = _tc_last_t(aggt, ht, ivl, Ws[nl - 1], Bs[nl - 1],
                        Wc1, bc1.reshape(_H, 1), Wc2, bc2.reshape(_C, 1))
    return probst[:, :_M].T.reshape(_B, _N, _C)


# speedup vs baseline: 1.1411x; 1.0055x over previous
import functools

import jax
import jax.numpy as jnp
from jax import lax
from jax.experimental import pallas as pl
from jax.experimental.pallas import tpu as pltpu
from jax.experimental.pallas import tpu_sc as plsc

_B, _N, _K, _D, _H, _C = 2, 10000, 16, 128, 128, 64
_M = _B * _N
_NW = 32
_MP = 20480
_HALF = _MP // 2
_CCH = 1024
_NCHK = _HALF // _CCH
_GRP = _CCH // 16
_W = _H // 2 // 16
_BN = 2048
_MASK = -65536


def _sc_aggregate_t(hTp, idxc):
    mesh = plsc.VectorSubcoreMesh(core_axis_name="c", subcore_axis_name="s")

    @functools.partial(
        pl.kernel,
        out_type=jax.ShapeDtypeStruct((_H, _MP), jnp.float32),
        mesh=mesh,
        scratch_types=[
            pltpu.VMEM((_W * _MP,), jnp.int32),
            pltpu.VMEM((_K, _CCH), jnp.int32),
            pltpu.VMEM((2 * _W, _CCH), jnp.float32),
        ],
        compiler_params=pltpu.CompilerParams(needs_layout_passes=False),
    )
    def agg_kernel(tab_hbm, idx_hbm, out_hbm, tab_v, idx_v, out_v):
        wid = lax.axis_index("s") * 2 + lax.axis_index("c")
        wr = (wid % 16) * _W
        half = wid // 16
        node0 = half * _HALF
        pltpu.sync_copy(tab_hbm.at[pl.ds(wr * _MP, _W * _MP)], tab_v)

        for chunk in range(_NCHK):
            pltpu.sync_copy(idx_hbm.at[half, chunk], idx_v)

            def group(g, carry):
                cols = [pl.ds(g * 32, 16), pl.ds(g * 32 + 16, 16)]
                acc_lo = [[None] * _W for _ in cols]
                acc_hi = [[None] * _W for _ in cols]
                for kk in range(_K):
                    for u, col in enumerate(cols):
                        iv = idx_v[kk, col]
                        for w in range(_W):
                            word = plsc.load_gather(tab_v, [iv + w * _MP])
                            lo = plsc.bitcast(word << 16, jnp.float32)
                            hi = plsc.bitcast(word & _MASK, jnp.float32)
                            if kk == 0:
                                acc_lo[u][w], acc_hi[u][w] = lo, hi
                            else:
                                acc_lo[u][w] = acc_lo[u][w] + lo
                                acc_hi[u][w] = acc_hi[u][w] + hi
                for u, col in enumerate(cols):
                    for w in range(_W):
                        out_v[w, col] = acc_lo[u][w]
                        out_v[_W + w, col] = acc_hi[u][w]
                return carry

            lax.fori_loop(0, _GRP // 2, group, 0)
            col0 = node0 + chunk * _CCH
            pltpu.sync_copy(
                out_v.at[pl.ds(0, _W)],
                out_hbm.at[pl.ds(wr, _W), pl.ds(col0, _CCH)])
            pltpu.sync_copy(
                out_v.at[pl.ds(_W, _W)],
                out_hbm.at[pl.ds(wr + 64, _W), pl.ds(col0, _CCH)])

    return agg_kernel(hTp, idxc)


def _pack_bf16_pairs(t):
    u = jax.lax.bitcast_convert_type(t, jnp.uint32)
    rne = lambda v: (v + jnp.uint32(0x7FFF) + ((v >> 16) & jnp.uint32(1))) >> 16
    word = rne(u[:64]) | (rne(u[64:]) << 16)
    return jax.lax.bitcast_convert_type(word, jnp.int32)


def _embed_body(x_ref, w_ref, ht_ref, pk_ref):
    x = x_ref[...]
    rs = jax.lax.dot_general(jnp.ones((1, _D), jnp.float32), x,
                             (((1,), (1,)), ((), ())),
                             preferred_element_type=jnp.float32)
    ht = jax.lax.dot_general(w_ref[...], x, (((0,), (1,)), ((), ())),
                             preferred_element_type=jnp.float32) / rs
    ht_ref[...] = ht
    pk_ref[...] = _pack_bf16_pairs(ht)


def _tc_embed_t(x, w):
    return pl.pallas_call(
        _embed_body,
        grid=(_MP // _BN,),
        in_specs=[
            pl.BlockSpec((_BN, _D), lambda i: (i, 0)),
            pl.BlockSpec((_D, _H), lambda i: (0, 0)),
        ],
        out_specs=[
            pl.BlockSpec((_H, _BN), lambda i: (0, i)),
            pl.BlockSpec((_H // 2, _BN), lambda i: (0, i)),
        ],
        out_shape=[
            jax.ShapeDtypeStruct((_H, _MP), jnp.float32),
            jax.ShapeDtypeStruct((_H // 2, _MP), jnp.int32),
        ],
    )(x, w)


def _layer_body(aggt_ref, ht_ref, ivl_ref, ws_ref, bs_ref, o_ref, pk_ref):
    a = aggt_ref[...] * ivl_ref[...]
    hn = jnp.maximum(
        jax.lax.dot_general(ws_ref[...], a, (((0,), (0,)), ((), ())),
                            preferred_element_type=jnp.float32)
        + jax.lax.dot_general(bs_ref[...], ht_ref[...],
                              (((0,), (0,)), ((), ())),
                              preferred_element_type=jnp.float32),
        0.0)
    o_ref[...] = hn
    pk_ref[...] = _pack_bf16_pairs(hn)


def _tc_layer_t(aggt, ht, ivl, ws, bs):
    return pl.pallas_call(
        _layer_body,
        grid=(_MP // _BN,),
        in_specs=[
            pl.BlockSpec((_H, _BN), lambda i: (0, i)),
            pl.BlockSpec((_H, _BN), lambda i: (0, i)),
            pl.BlockSpec((1, _BN), lambda i: (0, i)),
            pl.BlockSpec((_H, _H), lambda i: (0, 0)),
            pl.BlockSpec((_H, _H), lambda i: (0, 0)),
        ],
        out_specs=[
            pl.BlockSpec((_H, _BN), lambda i: (0, i)),
            pl.BlockSpec((_H // 2, _BN), lambda i: (0, i)),
        ],
        out_shape=[
            jax.ShapeDtypeStruct((_H, _MP), jnp.float32),
            jax.ShapeDtypeStruct((_H // 2, _MP), jnp.int32),
        ],
    )(aggt, ht, ivl, ws, bs)


def _last_body(aggt_ref, ht_ref, ivl_ref, ws_ref, bs_ref,
               w1_ref, b1_ref, w2_ref, b2_ref, o_ref):
    a = aggt_ref[...] * ivl_ref[...]
    hn = jnp.maximum(
        jax.lax.dot_general(ws_ref[...], a, (((0,), (0,)), ((), ())),
                            preferred_element_type=jnp.float32)
        + jax.lax.dot_general(bs_ref[...], ht_ref[...],
                              (((0,), (0,)), ((), ())),
                              preferred_element_type=jnp.float32),
        0.0)
    zt = jnp.maximum(
        jax.lax.dot_general(w1_ref[...], hn, (((0,), (0,)), ((), ())),
                            preferred_element_type=jnp.float32)
        + b1_ref[...],
        0.0)
    lg = jax.lax.dot_general(w2_ref[...], zt, (((0,), (0,)), ((), ())),
                             preferred_element_type=jnp.float32) + b2_ref[...]
    m = jnp.max(lg, axis=0, keepdims=True)
    e = jnp.exp(lg - m)
    o_ref[...] = e / jnp.sum(e, axis=0, keepdims=True)


def _tc_last_t(aggt, ht, ivl, ws, bs, w1, b1, w2, b2):
    return pl.pallas_call(
        _last_body,
        grid=(_MP // _BN,),
        in_specs=[
            pl.BlockSpec((_H, _BN), lambda i: (0, i)),
            pl.BlockSpec((_H, _BN), lambda i: (0, i)),
            pl.BlockSpec((1, _BN), lambda i: (0, i)),
            pl.BlockSpec((_H, _H), lambda i: (0, 0)),
            pl.BlockSpec((_H, _H), lambda i: (0, 0)),
            pl.BlockSpec((_H, _H), lambda i: (0, 0)),
            pl.BlockSpec((_H, 1), lambda i: (0, 0)),
            pl.BlockSpec((_H, _C), lambda i: (0, 0)),
            pl.BlockSpec((_C, 1), lambda i: (0, 0)),
        ],
        out_specs=pl.BlockSpec((_C, _BN), lambda i: (0, i)),
        out_shape=jax.ShapeDtypeStruct((_C, _MP), jnp.float32),
    )(aggt, ht, ivl, ws, bs, w1, b1, w2, b2)


def kernel(vertex_feat, neighbors_idx, valid_lens, W_embed, Ws, Bs,
           Wc1, bc1, Wc2, bc2):
    x = vertex_feat.reshape(_M, _D)
    xp = jnp.pad(x, ((0, _MP - _M), (0, 0)), constant_values=1.0)

    offs = (jnp.arange(_B, dtype=jnp.int32) * _N)[:, None, None]
    idx = (neighbors_idx + offs).reshape(_M, _K)
    idx = jnp.pad(idx, ((0, _MP - _M), (0, 0)))
    idxc = idx.T.reshape(_K, 2, _NCHK, _CCH).transpose(1, 2, 0, 3)

    vl = jnp.maximum(valid_lens, 1).astype(jnp.float32).reshape(1, _M)
    ivl = jnp.pad(1.0 / vl, ((0, 0), (0, _MP - _M)), constant_values=1.0)

    nl = Ws.shape[0]
    ht, htp = _tc_embed_t(xp, W_embed)
    for l in range(nl - 1):
        aggt = _sc_aggregate_t(htp.reshape(-1), idxc)
        ht, htp = _tc_layer_t(aggt, ht, ivl, Ws[l], Bs[l])
    aggt = _sc_aggregate_t(htp.reshape(-1), idxc)
    probst = _tc_last_t(aggt, ht, ivl, Ws[nl - 1], Bs[nl - 1],
                        Wc1, bc1.reshape(_H, 1), Wc2, bc2.reshape(_C, 1))
    return probst[:, :_M].T.reshape(_B, _N, _C)


# speedup vs baseline: 1.3015x; 1.1406x over previous
import functools

import jax
import jax.numpy as jnp
from jax import lax
from jax.experimental import pallas as pl
from jax.experimental.pallas import tpu as pltpu
from jax.experimental.pallas import tpu_sc as plsc

_B, _N, _K, _D, _H, _C = 2, 10000, 16, 128, 128, 64
_M = _B * _N
_NW = 32
_MP = 20480
_HALF = _MP // 2
_CCH = 640
_NCHK = _HALF // _CCH
_GRP = _CCH // 16
_W = _H // 2 // 16
_BN = 2048
_MASK = -65536


def _sc_aggregate_t(hTp, idxc):
    mesh = plsc.VectorSubcoreMesh(core_axis_name="c", subcore_axis_name="s")

    @functools.partial(
        pl.kernel,
        out_type=jax.ShapeDtypeStruct((_H, _MP), jnp.float32),
        mesh=mesh,
        scratch_types=[
            pltpu.VMEM((_W * _MP,), jnp.int32),
            pltpu.VMEM((2, _K, _CCH), jnp.int32),
            pltpu.VMEM((2, 2 * _W, _CCH), jnp.float32),
            pltpu.SemaphoreType.DMA,
            pltpu.SemaphoreType.DMA,
            pltpu.SemaphoreType.DMA,
        ],
        compiler_params=pltpu.CompilerParams(needs_layout_passes=False),
    )
    def agg_kernel(tab_hbm, idx_hbm, out_hbm, tab_v, idx_v, out_v,
                   sem_t, sem_i, sem_o):
        wid = lax.axis_index("s") * 2 + lax.axis_index("c")
        wr = (wid % 16) * _W
        half = wid // 16
        node0 = half * _HALF
        d_tab = pltpu.async_copy(
            tab_hbm.at[pl.ds(wr * _MP, _W * _MP)], tab_v, sem_t)
        d_idx = [None] * _NCHK
        d_out = [None] * _NCHK
        d_idx[0] = pltpu.async_copy(idx_hbm.at[half, 0], idx_v.at[0], sem_i)
        d_tab.wait()

        for chunk in range(_NCHK):
            b = chunk % 2
            d_idx[chunk].wait()
            if chunk + 1 < _NCHK:
                d_idx[chunk + 1] = pltpu.async_copy(
                    idx_hbm.at[half, chunk + 1], idx_v.at[1 - b], sem_i)
            if chunk >= 1:
                for d in d_out[chunk - 1]:
                    d.wait()

            def group(g, carry):
                col = pl.ds(g * 16, 16)
                acc_lo = [None] * _W
                acc_hi = [None] * _W
                for kk in range(_K):
                    iv = idx_v[b, kk, col]
                    for w in range(_W):
                        word = plsc.load_gather(tab_v, [iv + w * _MP])
                        lo = plsc.bitcast(word << 16, jnp.float32)
                        hi = plsc.bitcast(word & _MASK, jnp.float32)
                        if kk == 0:
                            acc_lo[w], acc_hi[w] = lo, hi
                        else:
                            acc_lo[w] = acc_lo[w] + lo
                            acc_hi[w] = acc_hi[w] + hi
                for w in range(_W):
                    out_v[b, w, col] = acc_lo[w]
                    out_v[b, _W + w, col] = acc_hi[w]
                return carry

            lax.fori_loop(0, _GRP, group, 0)
            col0 = node0 + chunk * _CCH
            d_out[chunk] = [
                pltpu.async_copy(
                    out_v.at[b, pl.ds(0, _W)],
                    out_hbm.at[pl.ds(wr, _W), pl.ds(col0, _CCH)], sem_o),
                pltpu.async_copy(
                    out_v.at[b, pl.ds(_W, _W)],
                    out_hbm.at[pl.ds(wr + 64, _W), pl.ds(col0, _CCH)], sem_o),
            ]
        for d in d_out[_NCHK - 1]:
            d.wait()

    return agg_kernel(hTp, idxc)


def _pack_bf16_pairs(t):
    u = jax.lax.bitcast_convert_type(t, jnp.uint32)
    rne = lambda v: (v + jnp.uint32(0x7FFF) + ((v >> 16) & jnp.uint32(1))) >> 16
    word = rne(u[:64]) | (rne(u[64:]) << 16)
    return jax.lax.bitcast_convert_type(word, jnp.int32)


def _embed_body(x_ref, w_ref, ht_ref, pk_ref):
    x = x_ref[...]
    rs = jax.lax.dot_general(jnp.ones((1, _D), jnp.float32), x,
                             (((1,), (1,)), ((), ())),
                             preferred_element_type=jnp.float32)
    ht = jax.lax.dot_general(w_ref[...], x, (((0,), (1,)), ((), ())),
                             preferred_element_type=jnp.float32) / rs
    ht_ref[...] = ht
    pk_ref[...] = _pack_bf16_pairs(ht)


def _tc_embed_t(x, w):
    return pl.pallas_call(
        _embed_body,
        grid=(_MP // _BN,),
        in_specs=[
            pl.BlockSpec((_BN, _D), lambda i: (i, 0)),
            pl.BlockSpec((_D, _H), lambda i: (0, 0)),
        ],
        out_specs=[
            pl.BlockSpec((_H, _BN), lambda i: (0, i)),
            pl.BlockSpec((_H // 2, _BN), lambda i: (0, i)),
        ],
        out_shape=[
            jax.ShapeDtypeStruct((_H, _MP), jnp.float32),
            jax.ShapeDtypeStruct((_H // 2, _MP), jnp.int32),
        ],
    )(x, w)


def _layer_body(aggt_ref, ht_ref, ivl_ref, ws_ref, bs_ref, o_ref, pk_ref):
    a = aggt_ref[...] * ivl_ref[...]
    hn = jnp.maximum(
        jax.lax.dot_general(ws_ref[...], a, (((0,), (0,)), ((), ())),
                            preferred_element_type=jnp.float32)
        + jax.lax.dot_general(bs_ref[...], ht_ref[...],
                              (((0,), (0,)), ((), ())),
                              preferred_element_type=jnp.float32),
        0.0)
    o_ref[...] = hn
    pk_ref[...] = _pack_bf16_pairs(hn)


def _tc_layer_t(aggt, ht, ivl, ws, bs):
    return pl.pallas_call(
        _layer_body,
        grid=(_MP // _BN,),
        in_specs=[
            pl.BlockSpec((_H, _BN), lambda i: (0, i)),
            pl.BlockSpec((_H, _BN), lambda i: (0, i)),
            pl.BlockSpec((1, _BN), lambda i: (0, i)),
            pl.BlockSpec((_H, _H), lambda i: (0, 0)),
            pl.BlockSpec((_H, _H), lambda i: (0, 0)),
        ],
        out_specs=[
            pl.BlockSpec((_H, _BN), lambda i: (0, i)),
            pl.BlockSpec((_H // 2, _BN), lambda i: (0, i)),
        ],
        out_shape=[
            jax.ShapeDtypeStruct((_H, _MP), jnp.float32),
            jax.ShapeDtypeStruct((_H // 2, _MP), jnp.int32),
        ],
    )(aggt, ht, ivl, ws, bs)


def _last_body(aggt_ref, ht_ref, ivl_ref, ws_ref, bs_ref,
               w1_ref, b1_ref, w2_ref, b2_ref, o_ref):
    a = aggt_ref[...] * ivl_ref[...]
    hn = jnp.maximum(
        jax.lax.dot_general(ws_ref[...], a, (((0,), (0,)), ((), ())),
                            preferred_element_type=jnp.float32)
        + jax.lax.dot_general(bs_ref[...], ht_ref[...],
                              (((0,), (0,)), ((), ())),
                              preferred_element_type=jnp.float32),
        0.0)
    zt = jnp.maximum(
        jax.lax.dot_general(w1_ref[...], hn, (((0,), (0,)), ((), ())),
                            preferred_element_type=jnp.float32)
        + b1_ref[...],
        0.0)
    lg = jax.lax.dot_general(w2_ref[...], zt, (((0,), (0,)), ((), ())),
                             preferred_element_type=jnp.float32) + b2_ref[...]
    m = jnp.max(lg, axis=0, keepdims=True)
    e = jnp.exp(lg - m)
    o_ref[...] = e / jnp.sum(e, axis=0, keepdims=True)


def _tc_last_t(aggt, ht, ivl, ws, bs, w1, b1, w2, b2):
    return pl.pallas_call(
        _last_body,
        grid=(_MP // _BN,),
        in_specs=[
            pl.BlockSpec((_H, _BN), lambda i: (0, i)),
            pl.BlockSpec((_H, _BN), lambda i: (0, i)),
            pl.BlockSpec((1, _BN), lambda i: (0, i)),
            pl.BlockSpec((_H, _H), lambda i: (0, 0)),
            pl.BlockSpec((_H, _H), lambda i: (0, 0)),
            pl.BlockSpec((_H, _H), lambda i: (0, 0)),
            pl.BlockSpec((_H, 1), lambda i: (0, 0)),
            pl.BlockSpec((_H, _C), lambda i: (0, 0)),
            pl.BlockSpec((_C, 1), lambda i: (0, 0)),
        ],
        out_specs=pl.BlockSpec((_C, _BN), lambda i: (0, i)),
        out_shape=jax.ShapeDtypeStruct((_C, _MP), jnp.float32),
    )(aggt, ht, ivl, ws, bs, w1, b1, w2, b2)


def kernel(vertex_feat, neighbors_idx, valid_lens, W_embed, Ws, Bs,
           Wc1, bc1, Wc2, bc2):
    x = vertex_feat.reshape(_M, _D)
    xp = jnp.pad(x, ((0, _MP - _M), (0, 0)), constant_values=1.0)

    offs = (jnp.arange(_B, dtype=jnp.int32) * _N)[:, None, None]
    idx = (neighbors_idx + offs).reshape(_M, _K)
    idx = jnp.pad(idx, ((0, _MP - _M), (0, 0)))
    idxc = idx.T.reshape(_K, 2, _NCHK, _CCH).transpose(1, 2, 0, 3)

    vl = jnp.maximum(valid_lens, 1).astype(jnp.float32).reshape(1, _M)
    ivl = jnp.pad(1.0 / vl, ((0, 0), (0, _MP - _M)), constant_values=1.0)

    nl = Ws.shape[0]
    ht, htp = _tc_embed_t(xp, W_embed)
    for l in range(nl - 1):
        aggt = _sc_aggregate_t(htp.reshape(-1), idxc)
        ht, htp = _tc_layer_t(aggt, ht, ivl, Ws[l], Bs[l])
    aggt = _sc_aggregate_t(htp.reshape(-1), idxc)
    probst = _tc_last_t(aggt, ht, ivl, Ws[nl - 1], Bs[nl - 1],
                        Wc1, bc1.reshape(_H, 1), Wc2, bc2.reshape(_C, 1))
    return probst[:, :_M].T.reshape(_B, _N, _C)


# speedup vs baseline: 1.3295x; 1.0215x over previous
import functools

import jax
import jax.numpy as jnp
from jax import lax
from jax.experimental import pallas as pl
from jax.experimental.pallas import tpu as pltpu
from jax.experimental.pallas import tpu_sc as plsc

_B, _N, _K, _D, _H, _C = 2, 10000, 16, 128, 128, 64
_M = _B * _N
_NW = 32
_MP = 20480
_HALF = _MP // 2
_CCH = 640
_NCHK = _HALF // _CCH
_GRP = _CCH // 16
_W = _H // 2 // 16
_BN = 4096
_MASK = -65536


def _sc_aggregate_t(hTp, idxc):
    mesh = plsc.VectorSubcoreMesh(core_axis_name="c", subcore_axis_name="s")

    @functools.partial(
        pl.kernel,
        out_type=jax.ShapeDtypeStruct((_H, _MP), jnp.float32),
        mesh=mesh,
        scratch_types=[
            pltpu.VMEM((_W * _MP,), jnp.int32),
            pltpu.VMEM((2, _K, _CCH), jnp.int32),
            pltpu.VMEM((2, 2 * _W, _CCH), jnp.float32),
            pltpu.SemaphoreType.DMA,
            pltpu.SemaphoreType.DMA,
            pltpu.SemaphoreType.DMA,
        ],
        compiler_params=pltpu.CompilerParams(needs_layout_passes=False),
    )
    def agg_kernel(tab_hbm, idx_hbm, out_hbm, tab_v, idx_v, out_v,
                   sem_t, sem_i, sem_o):
        wid = lax.axis_index("s") * 2 + lax.axis_index("c")
        wr = (wid % 16) * _W
        half = wid // 16
        node0 = half * _HALF
        d_tab = pltpu.async_copy(
            tab_hbm.at[pl.ds(wr * _MP, _W * _MP)], tab_v, sem_t)
        d_idx = [None] * _NCHK
        d_out = [None] * _NCHK
        d_idx[0] = pltpu.async_copy(idx_hbm.at[half, 0], idx_v.at[0], sem_i)
        d_tab.wait()

        for chunk in range(_NCHK):
            b = chunk % 2
            d_idx[chunk].wait()
            if chunk + 1 < _NCHK:
                d_idx[chunk + 1] = pltpu.async_copy(
                    idx_hbm.at[half, chunk + 1], idx_v.at[1 - b], sem_i)
            if chunk >= 1:
                for d in d_out[chunk - 1]:
                    d.wait()

            def group(g, carry):
                col = pl.ds(g * 16, 16)
                acc_lo = [None] * _W
                acc_hi = [None] * _W
                for kk in range(_K):
                    iv = idx_v[b, kk, col]
                    for w in range(_W):
                        word = plsc.load_gather(tab_v, [iv + w * _MP])
                        lo = plsc.bitcast(word << 16, jnp.float32)
                        hi = plsc.bitcast(word & _MASK, jnp.float32)
                        if kk == 0:
                            acc_lo[w], acc_hi[w] = lo, hi
                        else:
                            acc_lo[w] = acc_lo[w] + lo
                            acc_hi[w] = acc_hi[w] + hi
                for w in range(_W):
                    out_v[b, w, col] = acc_lo[w]
                    out_v[b, _W + w, col] = acc_hi[w]
                return carry

            lax.fori_loop(0, _GRP, group, 0)
            col0 = node0 + chunk * _CCH
            d_out[chunk] = [
                pltpu.async_copy(
                    out_v.at[b, pl.ds(0, _W)],
                    out_hbm.at[pl.ds(wr, _W), pl.ds(col0, _CCH)], sem_o),
                pltpu.async_copy(
                    out_v.at[b, pl.ds(_W, _W)],
                    out_hbm.at[pl.ds(wr + 64, _W), pl.ds(col0, _CCH)], sem_o),
            ]
        for d in d_out[_NCHK - 1]:
            d.wait()

    return agg_kernel(hTp, idxc)


def _pack_bf16_pairs(t):
    u = jax.lax.bitcast_convert_type(t, jnp.uint32)
    rne = lambda v: (v + jnp.uint32(0x7FFF) + ((v >> 16) & jnp.uint32(1))) >> 16
    word = rne(u[:64]) | (rne(u[64:]) << 16)
    return jax.lax.bitcast_convert_type(word, jnp.int32)


def _embed_body(x_ref, w_ref, ht_ref, pk_ref):
    x = x_ref[...]
    rs = jax.lax.dot_general(jnp.ones((1, _D), jnp.float32), x,
                             (((1,), (1,)), ((), ())),
                             preferred_element_type=jnp.float32)
    ht = jax.lax.dot_general(w_ref[...], x, (((0,), (1,)), ((), ())),
                             preferred_element_type=jnp.float32) / rs
    ht_ref[...] = ht
    pk_ref[...] = _pack_bf16_pairs(ht)


def _tc_embed_t(x, w):
    return pl.pallas_call(
        _embed_body,
        grid=(_MP // _BN,),
        in_specs=[
            pl.BlockSpec((_BN, _D), lambda i: (i, 0)),
            pl.BlockSpec((_D, _H), lambda i: (0, 0)),
        ],
        out_specs=[
            pl.BlockSpec((_H, _BN), lambda i: (0, i)),
            pl.BlockSpec((_H // 2, _BN), lambda i: (0, i)),
        ],
        out_shape=[
            jax.ShapeDtypeStruct((_H, _MP), jnp.float32),
            jax.ShapeDtypeStruct((_H // 2, _MP), jnp.int32),
        ],
    )(x, w)


def _layer_body(aggt_ref, ht_ref, ivl_ref, ws_ref, bs_ref, o_ref, pk_ref):
    a = aggt_ref[...] * ivl_ref[...]
    hn = jnp.maximum(
        jax.lax.dot_general(ws_ref[...], a, (((0,), (0,)), ((), ())),
                            preferred_element_type=jnp.float32)
        + jax.lax.dot_general(bs_ref[...], ht_ref[...],
                              (((0,), (0,)), ((), ())),
                              preferred_element_type=jnp.float32),
        0.0)
    o_ref[...] = hn
    pk_ref[...] = _pack_bf16_pairs(hn)


def _tc_layer_t(aggt, ht, ivl, ws, bs):
    return pl.pallas_call(
        _layer_body,
        grid=(_MP // _BN,),
        in_specs=[
            pl.BlockSpec((_H, _BN), lambda i: (0, i)),
            pl.BlockSpec((_H, _BN), lambda i: (0, i)),
            pl.BlockSpec((1, _BN), lambda i: (0, i)),
            pl.BlockSpec((_H, _H), lambda i: (0, 0)),
            pl.BlockSpec((_H, _H), lambda i: (0, 0)),
        ],
        out_specs=[
            pl.BlockSpec((_H, _BN), lambda i: (0, i)),
            pl.BlockSpec((_H // 2, _BN), lambda i: (0, i)),
        ],
        out_shape=[
            jax.ShapeDtypeStruct((_H, _MP), jnp.float32),
            jax.ShapeDtypeStruct((_H // 2, _MP), jnp.int32),
        ],
    )(aggt, ht, ivl, ws, bs)


def _last_body(aggt_ref, ht_ref, ivl_ref, ws_ref, bs_ref,
               w1_ref, b1_ref, w2_ref, b2_ref, o_ref):
    a = aggt_ref[...] * ivl_ref[...]
    hn = jnp.maximum(
        jax.lax.dot_general(ws_ref[...], a, (((0,), (0,)), ((), ())),
                            preferred_element_type=jnp.float32)
        + jax.lax.dot_general(bs_ref[...], ht_ref[...],
                              (((0,), (0,)), ((), ())),
                              preferred_element_type=jnp.float32),
        0.0)
    zt = jnp.maximum(
        jax.lax.dot_general(w1_ref[...], hn, (((0,), (0,)), ((), ())),
                            preferred_element_type=jnp.float32)
        + b1_ref[...],
        0.0)
    lg = jax.lax.dot_general(w2_ref[...], zt, (((0,), (0,)), ((), ())),
                             preferred_element_type=jnp.float32) + b2_ref[...]
    m = jnp.max(lg, axis=0, keepdims=True)
    e = jnp.exp(lg - m)
    o_ref[...] = e / jnp.sum(e, axis=0, keepdims=True)


def _tc_last_t(aggt, ht, ivl, ws, bs, w1, b1, w2, b2):
    return pl.pallas_call(
        _last_body,
        grid=(_MP // _BN,),
        in_specs=[
            pl.BlockSpec((_H, _BN), lambda i: (0, i)),
            pl.BlockSpec((_H, _BN), lambda i: (0, i)),
            pl.BlockSpec((1, _BN), lambda i: (0, i)),
            pl.BlockSpec((_H, _H), lambda i: (0, 0)),
            pl.BlockSpec((_H, _H), lambda i: (0, 0)),
            pl.BlockSpec((_H, _H), lambda i: (0, 0)),
            pl.BlockSpec((_H, 1), lambda i: (0, 0)),
            pl.BlockSpec((_H, _C), lambda i: (0, 0)),
            pl.BlockSpec((_C, 1), lambda i: (0, 0)),
        ],
        out_specs=pl.BlockSpec((_C, _BN), lambda i: (0, i)),
        out_shape=jax.ShapeDtypeStruct((_C, _MP), jnp.float32),
    )(aggt, ht, ivl, ws, bs, w1, b1, w2, b2)


def kernel(vertex_feat, neighbors_idx, valid_lens, W_embed, Ws, Bs,
           Wc1, bc1, Wc2, bc2):
    x = vertex_feat.reshape(_M, _D)
    xp = jnp.pad(x, ((0, _MP - _M), (0, 0)), constant_values=1.0)

    offs = (jnp.arange(_B, dtype=jnp.int32) * _N)[:, None, None]
    idx = (neighbors_idx + offs).reshape(_M, _K)
    idx = jnp.pad(idx, ((0, _MP - _M), (0, 0)))
    idxc = idx.T.reshape(_K, 2, _NCHK, _CCH).transpose(1, 2, 0, 3)

    vl = jnp.maximum(valid_lens, 1).astype(jnp.float32).reshape(1, _M)
    ivl = jnp.pad(1.0 / vl, ((0, 0), (0, _MP - _M)), constant_values=1.0)

    nl = Ws.shape[0]
    ht, htp = _tc_embed_t(xp, W_embed)
    for l in range(nl - 1):
        aggt = _sc_aggregate_t(htp.reshape(-1), idxc)
        ht, htp = _tc_layer_t(aggt, ht, ivl, Ws[l], Bs[l])
    aggt = _sc_aggregate_t(htp.reshape(-1), idxc)
    probst = _tc_last_t(aggt, ht, ivl, Ws[nl - 1], Bs[nl - 1],
                        Wc1, bc1.reshape(_H, 1), Wc2, bc2.reshape(_C, 1))
    return probst[:, :_M].T.reshape(_B, _N, _C)


# speedup vs baseline: 1.3503x; 1.0156x over previous
import functools

import jax
import jax.numpy as jnp
from jax import lax
from jax.experimental import pallas as pl
from jax.experimental.pallas import tpu as pltpu
from jax.experimental.pallas import tpu_sc as plsc

_B, _N, _K, _D, _H, _C = 2, 10000, 16, 128, 128, 64
_M = _B * _N
_NW = 32
_MP = 20480
_HALF = _MP // 2
_CCH = 640
_NCHK = _HALF // _CCH
_GRP = _CCH // 16
_W = _H // 2 // 16
_BN = 10240
_MASK = -65536


def _sc_aggregate_t(hTp, idxc):
    mesh = plsc.VectorSubcoreMesh(core_axis_name="c", subcore_axis_name="s")

    @functools.partial(
        pl.kernel,
        out_type=jax.ShapeDtypeStruct((_H, _MP), jnp.float32),
        mesh=mesh,
        scratch_types=[
            pltpu.VMEM((_W * _MP,), jnp.int32),
            pltpu.VMEM((2, _K, _CCH), jnp.int32),
            pltpu.VMEM((2, 2 * _W, _CCH), jnp.float32),
            pltpu.SemaphoreType.DMA,
            pltpu.SemaphoreType.DMA,
            pltpu.SemaphoreType.DMA,
        ],
        compiler_params=pltpu.CompilerParams(needs_layout_passes=False),
    )
    def agg_kernel(tab_hbm, idx_hbm, out_hbm, tab_v, idx_v, out_v,
                   sem_t, sem_i, sem_o):
        wid = lax.axis_index("s") * 2 + lax.axis_index("c")
        wr = (wid % 16) * _W
        half = wid // 16
        node0 = half * _HALF
        d_tab = pltpu.async_copy(
            tab_hbm.at[pl.ds(wr * _MP, _W * _MP)], tab_v, sem_t)
        d_idx = [None] * _NCHK
        d_out = [None] * _NCHK
        d_idx[0] = pltpu.async_copy(idx_hbm.at[half, 0], idx_v.at[0], sem_i)
        d_tab.wait()

        for chunk in range(_NCHK):
            b = chunk % 2
            d_idx[chunk].wait()
            if chunk + 1 < _NCHK:
                d_idx[chunk + 1] = pltpu.async_copy(
                    idx_hbm.at[half, chunk + 1], idx_v.at[1 - b], sem_i)
            if chunk >= 1:
                for d in d_out[chunk - 1]:
                    d.wait()

            def group(g, carry):
                col = pl.ds(g * 16, 16)
                acc_lo = [None] * _W
                acc_hi = [None] * _W
                for kk in range(_K):
                    iv = idx_v[b, kk, col]
                    for w in range(_W):
                        word = plsc.load_gather(tab_v, [iv + w * _MP])
                        lo = plsc.bitcast(word << 16, jnp.float32)
                        hi = plsc.bitcast(word & _MASK, jnp.float32)
                        if kk == 0:
                            acc_lo[w], acc_hi[w] = lo, hi
                        else:
                            acc_lo[w] = acc_lo[w] + lo
                            acc_hi[w] = acc_hi[w] + hi
                for w in range(_W):
                    out_v[b, w, col] = acc_lo[w]
                    out_v[b, _W + w, col] = acc_hi[w]
                return carry

            lax.fori_loop(0, _GRP, group, 0)
            col0 = node0 + chunk * _CCH
            d_out[chunk] = [
                pltpu.async_copy(
                    out_v.at[b, pl.ds(0, _W)],
                    out_hbm.at[pl.ds(wr, _W), pl.ds(col0, _CCH)], sem_o),
                pltpu.async_copy(
                    out_v.at[b, pl.ds(_W, _W)],
                    out_hbm.at[pl.ds(wr + 64, _W), pl.ds(col0, _CCH)], sem_o),
            ]
        for d in d_out[_NCHK - 1]:
            d.wait()

    return agg_kernel(hTp, idxc)


def _pack_bf16_pairs(t):
    u = jax.lax.bitcast_convert_type(t, jnp.uint32)
    rne = lambda v: (v + jnp.uint32(0x7FFF) + ((v >> 16) & jnp.uint32(1))) >> 16
    word = rne(u[:64]) | (rne(u[64:]) << 16)
    return jax.lax.bitcast_convert_type(word, jnp.int32)


def _embed_body(x_ref, w_ref, ht_ref, pk_ref):
    x = x_ref[...]
    rs = jax.lax.dot_general(jnp.ones((1, _D), jnp.float32), x,
                             (((1,), (1,)), ((), ())),
                             preferred_element_type=jnp.float32)
    ht = jax.lax.dot_general(w_ref[...], x, (((0,), (1,)), ((), ())),
                             preferred_element_type=jnp.float32) / rs
    ht_ref[...] = ht
    pk_ref[...] = _pack_bf16_pairs(ht)


def _tc_embed_t(x, w):
    return pl.pallas_call(
        _embed_body,
        grid=(_MP // _BN,),
        in_specs=[
            pl.BlockSpec((_BN, _D), lambda i: (i, 0)),
            pl.BlockSpec((_D, _H), lambda i: (0, 0)),
        ],
        out_specs=[
            pl.BlockSpec((_H, _BN), lambda i: (0, i)),
            pl.BlockSpec((_H // 2, _BN), lambda i: (0, i)),
        ],
        out_shape=[
            jax.ShapeDtypeStruct((_H, _MP), jnp.float32),
            jax.ShapeDtypeStruct((_H // 2, _MP), jnp.int32),
        ],
    )(x, w)


def _layer_body(aggt_ref, ht_ref, ivl_ref, ws_ref, bs_ref, o_ref, pk_ref):
    a = aggt_ref[...] * ivl_ref[...]
    hn = jnp.maximum(
        jax.lax.dot_general(ws_ref[...], a, (((0,), (0,)), ((), ())),
                            preferred_element_type=jnp.float32)
        + jax.lax.dot_general(bs_ref[...], ht_ref[...],
                              (((0,), (0,)), ((), ())),
                              preferred_element_type=jnp.float32),
        0.0)
    o_ref[...] = hn
    pk_ref[...] = _pack_bf16_pairs(hn)


def _tc_layer_t(aggt, ht, ivl, ws, bs):
    return pl.pallas_call(
        _layer_body,
        grid=(_MP // _BN,),
        in_specs=[
            pl.BlockSpec((_H, _BN), lambda i: (0, i)),
            pl.BlockSpec((_H, _BN), lambda i: (0, i)),
            pl.BlockSpec((1, _BN), lambda i: (0, i)),
            pl.BlockSpec((_H, _H), lambda i: (0, 0)),
            pl.BlockSpec((_H, _H), lambda i: (0, 0)),
        ],
        out_specs=[
            pl.BlockSpec((_H, _BN), lambda i: (0, i)),
            pl.BlockSpec((_H // 2, _BN), lambda i: (0, i)),
        ],
        out_shape=[
            jax.ShapeDtypeStruct((_H, _MP), jnp.float32),
            jax.ShapeDtypeStruct((_H // 2, _MP), jnp.int32),
        ],
    )(aggt, ht, ivl, ws, bs)


def _last_body(aggt_ref, ht_ref, ivl_ref, ws_ref, bs_ref,
               w1_ref, b1_ref, w2_ref, b2_ref, o_ref):
    a = aggt_ref[...] * ivl_ref[...]
    hn = jnp.maximum(
        jax.lax.dot_general(ws_ref[...], a, (((0,), (0,)), ((), ())),
                            preferred_element_type=jnp.float32)
        + jax.lax.dot_general(bs_ref[...], ht_ref[...],
                              (((0,), (0,)), ((), ())),
                              preferred_element_type=jnp.float32),
        0.0)
    zt = jnp.maximum(
        jax.lax.dot_general(w1_ref[...], hn, (((0,), (0,)), ((), ())),
                            preferred_element_type=jnp.float32)
        + b1_ref[...],
        0.0)
    lg = jax.lax.dot_general(w2_ref[...], zt, (((0,), (0,)), ((), ())),
                             preferred_element_type=jnp.float32) + b2_ref[...]
    m = jnp.max(lg, axis=0, keepdims=True)
    e = jnp.exp(lg - m)
    o_ref[...] = e / jnp.sum(e, axis=0, keepdims=True)


def _tc_last_t(aggt, ht, ivl, ws, bs, w1, b1, w2, b2):
    return pl.pallas_call(
        _last_body,
        grid=(_MP // _BN,),
        in_specs=[
            pl.BlockSpec((_H, _BN), lambda i: (0, i)),
            pl.BlockSpec((_H, _BN), lambda i: (0, i)),
            pl.BlockSpec((1, _BN), lambda i: (0, i)),
            pl.BlockSpec((_H, _H), lambda i: (0, 0)),
            pl.BlockSpec((_H, _H), lambda i: (0, 0)),
            pl.BlockSpec((_H, _H), lambda i: (0, 0)),
            pl.BlockSpec((_H, 1), lambda i: (0, 0)),
            pl.BlockSpec((_H, _C), lambda i: (0, 0)),
            pl.BlockSpec((_C, 1), lambda i: (0, 0)),
        ],
        out_specs=pl.BlockSpec((_C, _BN), lambda i: (0, i)),
        out_shape=jax.ShapeDtypeStruct((_C, _MP), jnp.float32),
    )(aggt, ht, ivl, ws, bs, w1, b1, w2, b2)


def kernel(vertex_feat, neighbors_idx, valid_lens, W_embed, Ws, Bs,
           Wc1, bc1, Wc2, bc2):
    x = vertex_feat.reshape(_M, _D)
    xp = jnp.pad(x, ((0, _MP - _M), (0, 0)), constant_values=1.0)

    offs = (jnp.arange(_B, dtype=jnp.int32) * _N)[:, None, None]
    idx = (neighbors_idx + offs).reshape(_M, _K)
    idx = jnp.pad(idx, ((0, _MP - _M), (0, 0)))
    idxc = idx.T.reshape(_K, 2, _NCHK, _CCH).transpose(1, 2, 0, 3)

    vl = jnp.maximum(valid_lens, 1).astype(jnp.float32).reshape(1, _M)
    ivl = jnp.pad(1.0 / vl, ((0, 0), (0, _MP - _M)), constant_values=1.0)

    nl = Ws.shape[0]
    ht, htp = _tc_embed_t(xp, W_embed)
    for l in range(nl - 1):
        aggt = _sc_aggregate_t(htp.reshape(-1), idxc)
        ht, htp = _tc_layer_t(aggt, ht, ivl, Ws[l], Bs[l])
    aggt = _sc_aggregate_t(htp.reshape(-1), idxc)
    probst = _tc_last_t(aggt, ht, ivl, Ws[nl - 1], Bs[nl - 1],
                        Wc1, bc1.reshape(_H, 1), Wc2, bc2.reshape(_C, 1))
    return probst[:, :_M].T.reshape(_B, _N, _C)
